# Initial kernel scaffold; baseline (speedup 1.0000x reference)
import functools

import jax
import jax.numpy as jnp
from jax import lax
from jax.experimental import pallas as pl
from jax.experimental.pallas import tpu as pltpu
from jax.experimental.pallas import tpu_sc as plsc

N = 10000
D = 128
H = 8
C = 16
FF = 512

NP = 10240
NW = 32
K = 128
E_TOT = 320000 + N
CHUNKS = -(-E_TOT // (NW * K))
EPW = CHUNKS * K
E_PAD = EPW * NW

RB1 = 2048
RB5 = 1024
RPT = NP // 16


def _dense_in_body(x_ref, w_ref, a_ref, xw_ref, ab_ref):
    xw = jnp.dot(x_ref[...], w_ref[...], preferred_element_type=jnp.float32)
    xw_ref[...] = xw
    ab_ref[...] = jnp.dot(xw, a_ref[...], preferred_element_type=jnp.float32)


def _dense_in(x_p, w_gat, a_cat):
    return pl.pallas_call(
        _dense_in_body,
        grid=(NP // RB1,),
        in_specs=[
            pl.BlockSpec((RB1, D), lambda i: (i, 0)),
            pl.BlockSpec((D, D), lambda i: (0, 0)),
            pl.BlockSpec((D, 2 * H), lambda i: (0, 0)),
        ],
        out_specs=[
            pl.BlockSpec((RB1, D), lambda i: (i, 0)),
            pl.BlockSpec((RB1, 2 * H), lambda i: (i, 0)),
        ],
        out_shape=[
            jax.ShapeDtypeStruct((NP, D), jnp.float32),
            jax.ShapeDtypeStruct((NP, 2 * H), jnp.float32),
        ],
    )(x_p, w_gat, a_cat)


def _edge_a_body(src_hbm, dst_hbm, ab_hbm, zer8_hbm,
                 s_hbm, den_hbm,
                 srcv, dstv, absrc, abdst, s2d, sem1, sem2, shared_den):
    cid = lax.axis_index("c")
    sid = lax.axis_index("s")
    wid = sid * 2 + cid
    pltpu.sync_copy(zer8_hbm.at[pl.ds(sid * RPT, RPT)],
                    shared_den.at[pl.ds(sid * RPT, RPT)])
    plsc.subcore_barrier()

    iota = lax.iota(jnp.int32, 16)
    c_vec = jnp.bitwise_and(iota, 7)
    r0 = jnp.right_shift(iota, 3)
    base0 = wid * EPW

    def chunk(ci, carry):
        base = base0 + ci * K
        pltpu.sync_copy(src_hbm.at[pl.ds(base, K)], srcv)
        pltpu.sync_copy(dst_hbm.at[pl.ds(base, K)], dstv)
        ga = pltpu.async_copy(ab_hbm.at[srcv], absrc, sem1)
        gb = pltpu.async_copy(ab_hbm.at[dstv], abdst, sem2)
        ga.wait()
        gb.wait()

        def inner(i, c2):
            r_vec = r0 + i * 2
            ea = (plsc.load_gather(absrc, [r_vec, c_vec])
                  + plsc.load_gather(abdst, [r_vec, c_vec + 8]))
            el = jnp.where(ea >= 0.0, ea, 0.2 * ea)
            plsc.store_scatter(s2d, [r_vec, c_vec], jnp.exp(el))
            return c2

        lax.fori_loop(0, K * H // 16, inner, 0)
        pltpu.sync_copy(s2d, s_hbm.at[pl.ds(base, K)])
        pltpu.sync_copy(s2d, shared_den.at[dstv], add=True)
        return carry

    lax.fori_loop(0, CHUNKS, chunk, 0)
    plsc.subcore_barrier()
    pltpu.sync_copy(shared_den.at[pl.ds(sid * RPT, RPT)],
                    den_hbm.at[pl.ds(cid * NP + sid * RPT, RPT)])


_edge_a = functools.partial(
    pl.kernel,
    out_type=[
        jax.ShapeDtypeStruct((E_PAD, H), jnp.float32),
        jax.ShapeDtypeStruct((2 * NP, H), jnp.float32),
    ],
    mesh=plsc.VectorSubcoreMesh(core_axis_name="c", subcore_axis_name="s"),
    scratch_types=[
        pltpu.VMEM((K,), jnp.int32),
        pltpu.VMEM((K,), jnp.int32),
        pltpu.VMEM((K, 2 * H), jnp.float32),
        pltpu.VMEM((K, 2 * H), jnp.float32),
        pltpu.VMEM((K, H), jnp.float32),
        pltpu.SemaphoreType.DMA,
        pltpu.SemaphoreType.DMA,
        pltpu.VMEM_SHARED((NP, H), jnp.float32),
    ],
)(_edge_a_body)


def _edge_b_body(src_hbm, dst_hbm, s_hbm, den_hbm, xw_hbm, zer128_hbm,
                 out_hbm,
                 srcv, dstv, dstv2, s2d, d0v, d1v, xwr,
                 sem1, sem2, sem3, shared_out):
    cid = lax.axis_index("c")
    sid = lax.axis_index("s")
    wid = sid * 2 + cid
    pltpu.sync_copy(zer128_hbm.at[pl.ds(sid * RPT, RPT)],
                    shared_out.at[pl.ds(sid * RPT, RPT)])
    plsc.subcore_barrier()

    iota = lax.iota(jnp.int32, 16)
    c_vec = jnp.bitwise_and(iota, 7)
    r0 = jnp.right_shift(iota, 3)
    base0 = wid * EPW

    def chunk(ci, carry):
        base = base0 + ci * K
        pltpu.sync_copy(src_hbm.at[pl.ds(base, K)], srcv)
        pltpu.sync_copy(dst_hbm.at[pl.ds(base, K)], dstv)

        def shift(i, c2):
            dstv2[pl.ds(i * 16, 16)] = dstv[pl.ds(i * 16, 16)] + NP
            return c2

        lax.fori_loop(0, K // 16, shift, 0)
        gx = pltpu.async_copy(xw_hbm.at[srcv], xwr, sem1)
        g0 = pltpu.async_copy(den_hbm.at[dstv], d0v, sem2)
        g1 = pltpu.async_copy(den_hbm.at[dstv2], d1v, sem3)
        pltpu.sync_copy(s_hbm.at[pl.ds(base, K)], s2d)
        g0.wait()
        g1.wait()

        def alpha(i, c2):
            r_vec = r0 + i * 2
            d = (plsc.load_gather(d0v, [r_vec, c_vec])
                 + plsc.load_gather(d1v, [r_vec, c_vec]))
            s = plsc.load_gather(s2d, [r_vec, c_vec])
            plsc.store_scatter(s2d, [r_vec, c_vec], s / (d + 1e-16))
            return c2

        lax.fori_loop(0, K * H // 16, alpha, 0)
        gx.wait()

        def scale(k, c2):
            rk = jnp.full((16,), k, jnp.int32)
            for h in range(H):
                colh = iota + h * C
                ah = plsc.load_gather(s2d, [rk, jnp.full((16,), h, jnp.int32)])
                v = plsc.load_gather(xwr, [rk, colh]) * ah
                plsc.store_scatter(xwr, [rk, colh], v)
            return c2

        lax.fori_loop(0, K, scale, 0)
        pltpu.sync_copy(xwr, shared_out.at[dstv], add=True)
        return carry

    lax.fori_loop(0, CHUNKS, chunk, 0)
    plsc.subcore_barrier()
    pltpu.sync_copy(shared_out.at[pl.ds(sid * RPT, RPT)],
                    out_hbm.at[pl.ds(cid * NP + sid * RPT, RPT)])


_edge_b = functools.partial(
    pl.kernel,
    out_type=[jax.ShapeDtypeStruct((2 * NP, D), jnp.float32)],
    mesh=plsc.VectorSubcoreMesh(core_axis_name="c", subcore_axis_name="s"),
    scratch_types=[
        pltpu.VMEM((K,), jnp.int32),
        pltpu.VMEM((K,), jnp.int32),
        pltpu.VMEM((K,), jnp.int32),
        pltpu.VMEM((K, H), jnp.float32),
        pltpu.VMEM((K, H), jnp.float32),
        pltpu.VMEM((K, H), jnp.float32),
        pltpu.VMEM((K, D), jnp.float32),
        pltpu.SemaphoreType.DMA,
        pltpu.SemaphoreType.DMA,
        pltpu.SemaphoreType.DMA,
        pltpu.VMEM_SHARED((NP, D), jnp.float32),
    ],
)(_edge_b_body)


def _dense_out_body(x_ref, p0_ref, p1_ref, bg_ref, w1_ref, b1_ref,
                    w2_ref, b2_ref, g1_ref, bt1_ref, g2_ref, bt2_ref,
                    out_ref):
    h_gat = p0_ref[...] + p1_ref[...] + bg_ref[...]
    t = x_ref[...] + h_gat
    mu = jnp.mean(t, axis=-1, keepdims=True)
    var = jnp.mean((t - mu) ** 2, axis=-1, keepdims=True)
    h1 = (t - mu) * lax.rsqrt(var + 1e-5) * g1_ref[...] + bt1_ref[...]
    m = jnp.dot(h1, w1_ref[...], preferred_element_type=jnp.float32) + b1_ref[...]
    m = jnp.maximum(m, 0.0)
    hf = jnp.dot(m, w2_ref[...], preferred_element_type=jnp.float32) + b2_ref[...]
    t2 = h1 + hf
    mu2 = jnp.mean(t2, axis=-1, keepdims=True)
    var2 = jnp.mean((t2 - mu2) ** 2, axis=-1, keepdims=True)
    out_ref[...] = ((t2 - mu2) * lax.rsqrt(var2 + 1e-5) * g2_ref[...]
                    + bt2_ref[...])


def _dense_out(x_p, parts, b_gat, w1, b1, w2, b2, g1, bt1, g2, bt2):
    full = lambda s: pl.BlockSpec(s, lambda i: (0, 0))
    return pl.pallas_call(
        _dense_out_body,
        grid=(NP // RB5,),
        in_specs=[
            pl.BlockSpec((RB5, D), lambda i: (i, 0)),
            pl.BlockSpec((RB5, D), lambda i: (i, 0)),
            pl.BlockSpec((RB5, D), lambda i: (i + NP // RB5, 0)),
            full((1, D)), full((D, FF)), full((1, FF)),
            full((FF, D)), full((1, D)), full((1, D)),
            full((1, D)), full((1, D)), full((1, D)),
        ],
        out_specs=pl.BlockSpec((RB5, D), lambda i: (i, 0)),
        out_shape=jax.ShapeDtypeStruct((NP, D), jnp.float32),
    )(x_p, parts, parts, b_gat.reshape(1, D), w1, b1.reshape(1, FF),
      w2, b2.reshape(1, D), g1.reshape(1, D), bt1.reshape(1, D),
      g2.reshape(1, D), bt2.reshape(1, D))


def kernel(x, edge_index, W_gat, att_src, att_dst, b_gat, W1, b1, W2, b2,
           g1, bt1, g2, bt2):
    n = x.shape[0]
    x_p = jnp.zeros((NP, D), jnp.float32).at[:n].set(x)

    ar = jnp.arange(n, dtype=edge_index.dtype)
    src = jnp.concatenate([edge_index[0], ar])
    dst = jnp.concatenate([edge_index[1], ar])
    pad = jnp.full((E_PAD - E_TOT,), n, src.dtype)
    src = jnp.concatenate([src, pad]).astype(jnp.int32)
    dst = jnp.concatenate([dst, pad]).astype(jnp.int32)

    eye = jnp.eye(H, dtype=jnp.float32)
    a_s = (att_src[:, :, None] * eye[:, None, :]).reshape(D, H)
    a_d = (att_dst[:, :, None] * eye[:, None, :]).reshape(D, H)
    a_cat = jnp.concatenate([a_s, a_d], axis=1)

    xw, ab = _dense_in(x_p, W_gat, a_cat)
    zer8 = jnp.zeros((NP, H), jnp.float32)
    zer128 = jnp.zeros((NP, D), jnp.float32)

    s_e, den = _edge_a(src, dst, ab, zer8)
    (parts,) = _edge_b(src, dst, s_e, den, xw, zer128)

    out = _dense_out(x_p, parts, b_gat, W1, b1, W2, b2, g1, bt1, g2, bt2)
    return out[:n]


# speedup vs baseline: 36.9530x; 36.9530x over previous
import functools

import jax
import jax.numpy as jnp
from jax import lax
from jax.experimental import pallas as pl
from jax.experimental.pallas import tpu as pltpu
from jax.experimental.pallas import tpu_sc as plsc

N = 10000
D = 128
H = 8
C = 16
FF = 512

NP = 10240
NW = 32
K = 128
E_TOT = 320000 + N
CHUNKS = -(-E_TOT // (NW * K))
EPW = CHUNKS * K
E_PAD = EPW * NW

RB1 = 2048
RB5 = 1024
RPT = NP // 16


def _dense_in_body(x_ref, w_ref, a_ref, xw_ref, ab_ref):
    xw = jnp.dot(x_ref[...], w_ref[...], preferred_element_type=jnp.float32)
    xw_ref[...] = xw
    ab_ref[...] = jnp.dot(xw, a_ref[...], preferred_element_type=jnp.float32)


def _dense_in(x_p, w_gat, a_cat):
    return pl.pallas_call(
        _dense_in_body,
        grid=(NP // RB1,),
        in_specs=[
            pl.BlockSpec((RB1, D), lambda i: (i, 0)),
            pl.BlockSpec((D, D), lambda i: (0, 0)),
            pl.BlockSpec((D, 2 * H), lambda i: (0, 0)),
        ],
        out_specs=[
            pl.BlockSpec((RB1, D), lambda i: (i, 0)),
            pl.BlockSpec((RB1, 2 * H), lambda i: (i, 0)),
        ],
        out_shape=[
            jax.ShapeDtypeStruct((NP, D), jnp.float32),
            jax.ShapeDtypeStruct((NP, 2 * H), jnp.float32),
        ],
    )(x_p, w_gat, a_cat)


def _edge_a_body(src_hbm, dst_hbm, ab_hbm, zer8_hbm,
                 s_hbm, den_hbm,
                 srcv, dstv, absrc, abdst, s2d, sem1, sem2, shared_den):
    cid = lax.axis_index("c")
    sid = lax.axis_index("s")
    wid = sid * 2 + cid
    pltpu.sync_copy(zer8_hbm.at[pl.ds(sid * RPT, RPT)],
                    shared_den.at[pl.ds(sid * RPT, RPT)])
    plsc.subcore_barrier()

    iota = lax.iota(jnp.int32, 16)
    c_vec = jnp.bitwise_and(iota, 7)
    r0 = jnp.right_shift(iota, 3)
    base0 = wid * EPW

    def chunk(ci, carry):
        base = base0 + ci * K
        pltpu.sync_copy(src_hbm.at[pl.ds(base, K)], srcv)
        pltpu.sync_copy(dst_hbm.at[pl.ds(base, K)], dstv)
        ga = pltpu.async_copy(ab_hbm.at[srcv], absrc, sem1)
        gb = pltpu.async_copy(ab_hbm.at[dstv], abdst, sem2)
        ga.wait()
        gb.wait()

        def inner(i, c2):
            r_vec = r0 + i * 2
            ea = (plsc.load_gather(absrc, [r_vec, c_vec])
                  + plsc.load_gather(abdst, [r_vec, c_vec + 8]))
            el = jnp.where(ea >= 0.0, ea, 0.2 * ea)
            plsc.store_scatter(s2d, [r_vec, c_vec], jnp.exp(el))
            return c2

        lax.fori_loop(0, K * H // 16, inner, 0)
        pltpu.sync_copy(s2d, s_hbm.at[pl.ds(base, K)])
        pltpu.sync_copy(s2d, shared_den.at[dstv], add=True)
        return carry

    lax.fori_loop(0, CHUNKS, chunk, 0)
    plsc.subcore_barrier()
    pltpu.sync_copy(shared_den.at[pl.ds(sid * RPT, RPT)],
                    den_hbm.at[pl.ds(cid * NP + sid * RPT, RPT)])


_edge_a = functools.partial(
    pl.kernel,
    out_type=[
        jax.ShapeDtypeStruct((E_PAD, H), jnp.float32),
        jax.ShapeDtypeStruct((2 * NP, H), jnp.float32),
    ],
    mesh=plsc.VectorSubcoreMesh(core_axis_name="c", subcore_axis_name="s", num_cores=2, num_subcores=16),
    compiler_params=pltpu.CompilerParams(needs_layout_passes=False, use_tc_tiling_on_sc=False),
    scratch_types=[
        pltpu.VMEM((K,), jnp.int32),
        pltpu.VMEM((K,), jnp.int32),
        pltpu.VMEM((K, 2 * H), jnp.float32),
        pltpu.VMEM((K, 2 * H), jnp.float32),
        pltpu.VMEM((K, H), jnp.float32),
        pltpu.SemaphoreType.DMA,
        pltpu.SemaphoreType.DMA,
        pltpu.VMEM_SHARED((NP, H), jnp.float32),
    ],
)(_edge_a_body)


def _edge_b_body(src_hbm, dst_hbm, s_hbm, den_hbm, xw_hbm, zer128_hbm,
                 out_hbm,
                 srcv, dstv, dstv2, s2d, d0v, d1v, xwr,
                 sem1, sem2, sem3, shared_out):
    cid = lax.axis_index("c")
    sid = lax.axis_index("s")
    wid = sid * 2 + cid
    pltpu.sync_copy(zer128_hbm.at[pl.ds(sid * RPT, RPT)],
                    shared_out.at[pl.ds(sid * RPT, RPT)])
    plsc.subcore_barrier()

    iota = lax.iota(jnp.int32, 16)
    c_vec = jnp.bitwise_and(iota, 7)
    r0 = jnp.right_shift(iota, 3)
    base0 = wid * EPW

    def chunk(ci, carry):
        base = base0 + ci * K
        pltpu.sync_copy(src_hbm.at[pl.ds(base, K)], srcv)
        pltpu.sync_copy(dst_hbm.at[pl.ds(base, K)], dstv)

        def shift(i, c2):
            dstv2[pl.ds(i * 16, 16)] = dstv[pl.ds(i * 16, 16)] + NP
            return c2

        lax.fori_loop(0, K // 16, shift, 0)
        gx = pltpu.async_copy(xw_hbm.at[srcv], xwr, sem1)
        g0 = pltpu.async_copy(den_hbm.at[dstv], d0v, sem2)
        g1 = pltpu.async_copy(den_hbm.at[dstv2], d1v, sem3)
        pltpu.sync_copy(s_hbm.at[pl.ds(base, K)], s2d)
        g0.wait()
        g1.wait()

        def alpha(i, c2):
            r_vec = r0 + i * 2
            d = (plsc.load_gather(d0v, [r_vec, c_vec])
                 + plsc.load_gather(d1v, [r_vec, c_vec]))
            s = plsc.load_gather(s2d, [r_vec, c_vec])
            plsc.store_scatter(s2d, [r_vec, c_vec], s / (d + 1e-16))
            return c2

        lax.fori_loop(0, K * H // 16, alpha, 0)
        gx.wait()

        def scale(k, c2):
            rk = jnp.full((16,), k, jnp.int32)
            for h in range(H):
                colh = iota + h * C
                ah = plsc.load_gather(s2d, [rk, jnp.full((16,), h, jnp.int32)])
                v = plsc.load_gather(xwr, [rk, colh]) * ah
                plsc.store_scatter(xwr, [rk, colh], v)
            return c2

        lax.fori_loop(0, K, scale, 0)
        pltpu.sync_copy(xwr, shared_out.at[dstv], add=True)
        return carry

    lax.fori_loop(0, CHUNKS, chunk, 0)
    plsc.subcore_barrier()
    pltpu.sync_copy(shared_out.at[pl.ds(sid * RPT, RPT)],
                    out_hbm.at[pl.ds(cid * NP + sid * RPT, RPT)])


_edge_b = functools.partial(
    pl.kernel,
    out_type=[jax.ShapeDtypeStruct((2 * NP, D), jnp.float32)],
    mesh=plsc.VectorSubcoreMesh(core_axis_name="c", subcore_axis_name="s", num_cores=2, num_subcores=16),
    compiler_params=pltpu.CompilerParams(needs_layout_passes=False, use_tc_tiling_on_sc=False),
    scratch_types=[
        pltpu.VMEM((K,), jnp.int32),
        pltpu.VMEM((K,), jnp.int32),
        pltpu.VMEM((K,), jnp.int32),
        pltpu.VMEM((K, H), jnp.float32),
        pltpu.VMEM((K, H), jnp.float32),
        pltpu.VMEM((K, H), jnp.float32),
        pltpu.VMEM((K, D), jnp.float32),
        pltpu.SemaphoreType.DMA,
        pltpu.SemaphoreType.DMA,
        pltpu.SemaphoreType.DMA,
        pltpu.VMEM_SHARED((NP, D), jnp.float32),
    ],
)(_edge_b_body)


def _dense_out_body(x_ref, p0_ref, p1_ref, bg_ref, w1_ref, b1_ref,
                    w2_ref, b2_ref, g1_ref, bt1_ref, g2_ref, bt2_ref,
                    out_ref):
    h_gat = p0_ref[...] + p1_ref[...] + bg_ref[...]
    t = x_ref[...] + h_gat
    mu = jnp.mean(t, axis=-1, keepdims=True)
    var = jnp.mean((t - mu) ** 2, axis=-1, keepdims=True)
    h1 = (t - mu) * lax.rsqrt(var + 1e-5) * g1_ref[...] + bt1_ref[...]
    m = jnp.dot(h1, w1_ref[...], preferred_element_type=jnp.float32) + b1_ref[...]
    m = jnp.maximum(m, 0.0)
    hf = jnp.dot(m, w2_ref[...], preferred_element_type=jnp.float32) + b2_ref[...]
    t2 = h1 + hf
    mu2 = jnp.mean(t2, axis=-1, keepdims=True)
    var2 = jnp.mean((t2 - mu2) ** 2, axis=-1, keepdims=True)
    out_ref[...] = ((t2 - mu2) * lax.rsqrt(var2 + 1e-5) * g2_ref[...]
                    + bt2_ref[...])


def _dense_out(x_p, parts, b_gat, w1, b1, w2, b2, g1, bt1, g2, bt2):
    full = lambda s: pl.BlockSpec(s, lambda i: (0, 0))
    return pl.pallas_call(
        _dense_out_body,
        grid=(NP // RB5,),
        in_specs=[
            pl.BlockSpec((RB5, D), lambda i: (i, 0)),
            pl.BlockSpec((RB5, D), lambda i: (i, 0)),
            pl.BlockSpec((RB5, D), lambda i: (i + NP // RB5, 0)),
            full((1, D)), full((D, FF)), full((1, FF)),
            full((FF, D)), full((1, D)), full((1, D)),
            full((1, D)), full((1, D)), full((1, D)),
        ],
        out_specs=pl.BlockSpec((RB5, D), lambda i: (i, 0)),
        out_shape=jax.ShapeDtypeStruct((NP, D), jnp.float32),
    )(x_p, parts, parts, b_gat.reshape(1, D), w1, b1.reshape(1, FF),
      w2, b2.reshape(1, D), g1.reshape(1, D), bt1.reshape(1, D),
      g2.reshape(1, D), bt2.reshape(1, D))


def kernel(x, edge_index, W_gat, att_src, att_dst, b_gat, W1, b1, W2, b2,
           g1, bt1, g2, bt2):
    n = x.shape[0]
    x_p = jnp.zeros((NP, D), jnp.float32).at[:n].set(x)

    ar = jnp.arange(n, dtype=edge_index.dtype)
    src = jnp.concatenate([edge_index[0], ar])
    dst = jnp.concatenate([edge_index[1], ar])
    pad = jnp.full((E_PAD - E_TOT,), n, src.dtype)
    src = jnp.concatenate([src, pad]).astype(jnp.int32)
    dst = jnp.concatenate([dst, pad]).astype(jnp.int32)

    eye = jnp.eye(H, dtype=jnp.float32)
    a_s = (att_src[:, :, None] * eye[:, None, :]).reshape(D, H)
    a_d = (att_dst[:, :, None] * eye[:, None, :]).reshape(D, H)
    a_cat = jnp.concatenate([a_s, a_d], axis=1)

    xw, ab = _dense_in(x_p, W_gat, a_cat)
    zer8 = jnp.zeros((NP, H), jnp.float32)
    zer128 = jnp.zeros((NP, D), jnp.float32)

    s_e, den = _edge_a(src, dst, ab, zer8)
    (parts,) = _edge_b(src, dst, s_e, den, xw, zer128)

    out = _dense_out(x_p, parts, b_gat, W1, b1, W2, b2, g1, bt1, g2, bt2)
    return out[:n]
